# baseline (speedup 1.0000x reference)
import functools

import jax
import jax.numpy as jnp
from jax import lax
from jax.experimental import pallas as pl
from jax.experimental.pallas import tpu as pltpu
from jax.experimental.pallas import tpu_sc as plsc

_NC, _NS = 2, 16
_NW = _NC * _NS
_B = 16384
_D = 32
_U = 100001
_HALF = 50000
_PBLK = 400
_NPB = _HALF // _PBLK
_BROWS = 784
_BPW = _B // _NW
_CHUNK = 128
_NCH = _BPW // _CHUNK


def _pack_body(gl, ml, gh, mh, out):
    out[...] = jnp.concatenate([gl[...], ml[...], gh[...], mh[...]], axis=1)


_pack_lo = pl.BlockSpec((_PBLK, _D), lambda i: (i, 0))
_pack_hi = pl.BlockSpec((_PBLK, _D), lambda i: (i + _NPB, 0))

_pack = pl.pallas_call(
    _pack_body,
    grid=(_NPB,),
    in_specs=[_pack_lo, _pack_lo, _pack_hi, _pack_hi],
    out_specs=pl.BlockSpec((_PBLK, 128), lambda i: (i, 0)),
    out_shape=jax.ShapeDtypeStruct((_HALF, 128), jnp.float32),
)


def _sc_body(uid, iid, p_u, p_i, b_u, b_i,
             o_u, o_i, o_ub, o_ib,
             idx, idx2, buf, bias_v, sem, sem2):
    cid = lax.axis_index("c")
    sid = lax.axis_index("s")
    base = (cid * _NS + sid) * _BPW

    for ids_hbm, p, b, o_rows, o_b in ((uid, p_u, b_u, o_u, o_ub),
                                       (iid, p_i, b_i, o_i, o_ib)):
        for c in range(_NCH):
            pltpu.sync_copy(ids_hbm.at[pl.ds(base + c * _CHUNK, _CHUNK)],
                            idx.at[c])
        for c in range(_NCH):
            for k in range(_CHUNK // 16):
                sl = pl.ds(16 * k, 16)
                v = idx[c, sl]
                idx2[c, sl] = jnp.where(v < _HALF, v, v - _HALF)
        gs = [pltpu.async_copy(p.at[idx2.at[c]], buf.at[c], sem)
              for c in range(_NCH)]
        bs = [pltpu.async_copy(b.at[idx.at[c]], bias_v.at[c], sem)
              for c in range(_NCH)]
        ws = []
        for c in range(_NCH):
            gs[c].wait()
            ws.append(pltpu.async_copy(
                buf.at[c], o_rows.at[pl.ds(base + c * _CHUNK, _CHUNK)], sem2))
        for c in range(_NCH):
            bs[c].wait()
            ws.append(pltpu.async_copy(
                bias_v.at[c], o_b.at[pl.ds(base + c * _CHUNK, _CHUNK)], sem2))
        for w in ws:
            w.wait()


@functools.cache
def _make_sc_gather():
    return pl.kernel(
        _sc_body,
        out_type=[
            jax.ShapeDtypeStruct((_B, 128), jnp.float32),
            jax.ShapeDtypeStruct((_B, 128), jnp.float32),
            jax.ShapeDtypeStruct((_B,), jnp.float32),
            jax.ShapeDtypeStruct((_B,), jnp.float32),
        ],
        mesh=plsc.VectorSubcoreMesh(
            core_axis_name="c", subcore_axis_name="s",
            num_cores=_NC, num_subcores=_NS),
        scratch_types=[
            pltpu.VMEM((_NCH, _CHUNK), jnp.int32),
            pltpu.VMEM((_NCH, _CHUNK), jnp.int32),
            pltpu.VMEM((_NCH, _CHUNK, 128), jnp.float32),
            pltpu.VMEM((_NCH, _CHUNK), jnp.float32),
            pltpu.SemaphoreType.DMA,
            pltpu.SemaphoreType.DMA,
        ],
        compiler_params=pltpu.CompilerParams(use_tc_tiling_on_sc=True,
                                             needs_layout_passes=False),
    )


_BLK = 2048
_NBLK = _B // _BLK


def _mm(a, b):
    return lax.dot_general(a, b, (((1,), (1,)), ((), ())),
                           preferred_element_type=jnp.float32)


def _tc_body(pu, pi, uidr, iidr, ub, ib, w1, b1, w2, b2, wf, gb, bfs, out):
    pur = pu[...]
    pir = pi[...]
    su = uidr[...] < _HALF
    si = iidr[...] < _HALF
    gu = jnp.where(su, pur[:, 0:_D], pur[:, 64:64 + _D])
    mu = jnp.where(su, pur[:, _D:2 * _D], pur[:, 64 + _D:128])
    gi = jnp.where(si, pir[:, 0:_D], pir[:, 64:64 + _D])
    mi = jnp.where(si, pir[:, _D:2 * _D], pir[:, 64 + _D:128])
    w1v = w1[...]
    wfv = wf[...]
    h = _mm(mu, w1v[:, :_D]) + _mm(mi, w1v[:, _D:]) + b1[...]
    h = jnp.maximum(h, 0.0)
    h = _mm(h, w2[...]) + b2[...]
    r = jnp.sum(gu * gi * wfv[:, :_D], axis=1, keepdims=True)
    r = r + jnp.sum(h * wfv[:, _D:], axis=1, keepdims=True)
    out[...] = ub[...] + ib[...] + r + (gb[0, 0] + bfs[0, 0])


_prow_spec = pl.BlockSpec((_BLK, 128), lambda i: (i, 0))
_col_spec = pl.BlockSpec((_BLK, 1), lambda i: (i, 0))
_full = lambda s: pl.BlockSpec(s, lambda i: (0,) * len(s))

_tc_dense = pl.pallas_call(
    _tc_body,
    grid=(_NBLK,),
    in_specs=[
        _prow_spec,
        _prow_spec,
        _col_spec,
        _col_spec,
        _col_spec,
        _col_spec,
        _full((_D, 2 * _D)),
        _full((1, _D)),
        _full((_D, _D)),
        _full((1, _D)),
        _full((1, 2 * _D)),
        _full((1, 1)),
        _full((1, 1)),
    ],
    out_specs=_col_spec,
    out_shape=jax.ShapeDtypeStruct((_B, 1), jnp.float32),
)


def kernel(d0, d1, d2, d3, d4, user_id, item_id, user_bias, item_bias,
           global_bias, gmf_user_emb, gmf_item_emb, mlp_user_emb, mlp_item_emb,
           W1, b1, W2, b2, Wf, bf):
    p_u = _pack(gmf_user_emb, mlp_user_emb, gmf_user_emb, mlp_user_emb)
    p_i = _pack(gmf_item_emb, mlp_item_emb, gmf_item_emb, mlp_item_emb)
    o_u, o_i, ub, ib = _make_sc_gather()(user_id, item_id, p_u, p_i,
                                         user_bias, item_bias)
    out = _tc_dense(
        o_u, o_i, user_id.reshape(_B, 1), item_id.reshape(_B, 1),
        ub.reshape(_B, 1), ib.reshape(_B, 1),
        W1, b1.reshape(1, _D), W2, b2.reshape(1, _D), Wf,
        global_bias.reshape(1, 1), bf.reshape(1, 1))
    return out[:, 0]


# speedup vs baseline: 1.6096x; 1.6096x over previous
import functools

import jax
import jax.numpy as jnp
from jax import lax
from jax.experimental import pallas as pl
from jax.experimental.pallas import tpu as pltpu
from jax.experimental.pallas import tpu_sc as plsc

_NC, _NS = 2, 16
_NW = _NC * _NS
_B = 16384
_D = 32
_U = 100001
_HALF = 50176
_PBLK = 512
_NPB = _HALF // _PBLK
_BROWS = 784
_BPW = _B // _NW
_CHUNK = 128
_NCH = _BPW // _CHUNK


def _pack_body(gl, ml, gh, mh, out):
    out[...] = jnp.concatenate(
        [gl[...].T, ml[...].T, gh[...].T, mh[...].T], axis=1)


_pack_lo = pl.BlockSpec((_D, _PBLK), lambda i: (0, i))
_pack_hi = pl.BlockSpec((_D, _PBLK), lambda i: (0, i + _NPB))

_pack = pl.pallas_call(
    _pack_body,
    grid=(_NPB,),
    in_specs=[_pack_lo, _pack_lo, _pack_hi, _pack_hi],
    out_specs=pl.BlockSpec((_PBLK, 128), lambda i: (i, 0)),
    out_shape=jax.ShapeDtypeStruct((_HALF, 128), jnp.float32),
)


def _sc_body(uid, iid, p_u, p_i, b_u, b_i,
             o_u, o_i, o_ub, o_ib,
             idx, idx2, buf, bias_v, sem, sem2):
    cid = lax.axis_index("c")
    sid = lax.axis_index("s")
    base = (cid * _NS + sid) * _BPW

    for ids_hbm, p, b, o_rows, o_b in ((uid, p_u, b_u, o_u, o_ub),
                                       (iid, p_i, b_i, o_i, o_ib)):
        for c in range(_NCH):
            pltpu.sync_copy(ids_hbm.at[pl.ds(base + c * _CHUNK, _CHUNK)],
                            idx.at[c])
        for c in range(_NCH):
            for k in range(_CHUNK // 16):
                sl = pl.ds(16 * k, 16)
                v = idx[c, sl]
                idx2[c, sl] = jnp.where(v < _HALF, v, v - _HALF)
        gs = [pltpu.async_copy(p.at[idx2.at[c]], buf.at[c], sem)
              for c in range(_NCH)]
        bs = [pltpu.async_copy(b.at[idx.at[c]], bias_v.at[c], sem)
              for c in range(_NCH)]
        ws = []
        for c in range(_NCH):
            gs[c].wait()
            ws.append(pltpu.async_copy(
                buf.at[c], o_rows.at[pl.ds(base + c * _CHUNK, _CHUNK)], sem2))
        for c in range(_NCH):
            bs[c].wait()
            ws.append(pltpu.async_copy(
                bias_v.at[c], o_b.at[pl.ds(base + c * _CHUNK, _CHUNK)], sem2))
        for w in ws:
            w.wait()


@functools.cache
def _make_sc_gather():
    return pl.kernel(
        _sc_body,
        out_type=[
            jax.ShapeDtypeStruct((_B, 128), jnp.float32),
            jax.ShapeDtypeStruct((_B, 128), jnp.float32),
            jax.ShapeDtypeStruct((_B,), jnp.float32),
            jax.ShapeDtypeStruct((_B,), jnp.float32),
        ],
        mesh=plsc.VectorSubcoreMesh(
            core_axis_name="c", subcore_axis_name="s",
            num_cores=_NC, num_subcores=_NS),
        scratch_types=[
            pltpu.VMEM((_NCH, _CHUNK), jnp.int32),
            pltpu.VMEM((_NCH, _CHUNK), jnp.int32),
            pltpu.VMEM((_NCH, _CHUNK, 128), jnp.float32),
            pltpu.VMEM((_NCH, _CHUNK), jnp.float32),
            pltpu.SemaphoreType.DMA,
            pltpu.SemaphoreType.DMA,
        ],
        compiler_params=pltpu.CompilerParams(use_tc_tiling_on_sc=True,
                                             needs_layout_passes=False),
    )


_BLK = 2048
_NBLK = _B // _BLK


def _mm(a, b):
    return lax.dot_general(a, b, (((1,), (1,)), ((), ())),
                           preferred_element_type=jnp.float32)


def _tc_body(pu, pi, uidr, iidr, ub, ib, w1, b1, w2, b2, wf, gb, bfs, out):
    pur = pu[...]
    pir = pi[...]
    su = uidr[...] < _HALF
    si = iidr[...] < _HALF
    gu = jnp.where(su, pur[:, 0:_D], pur[:, 64:64 + _D])
    mu = jnp.where(su, pur[:, _D:2 * _D], pur[:, 64 + _D:128])
    gi = jnp.where(si, pir[:, 0:_D], pir[:, 64:64 + _D])
    mi = jnp.where(si, pir[:, _D:2 * _D], pir[:, 64 + _D:128])
    w1v = w1[...]
    wfv = wf[...]
    h = _mm(mu, w1v[:, :_D]) + _mm(mi, w1v[:, _D:]) + b1[...]
    h = jnp.maximum(h, 0.0)
    h = _mm(h, w2[...]) + b2[...]
    r = jnp.sum(gu * gi * wfv[:, :_D], axis=1, keepdims=True)
    r = r + jnp.sum(h * wfv[:, _D:], axis=1, keepdims=True)
    out[...] = ub[...] + ib[...] + r + (gb[0, 0] + bfs[0, 0])


_prow_spec = pl.BlockSpec((_BLK, 128), lambda i: (i, 0))
_col_spec = pl.BlockSpec((_BLK, 1), lambda i: (i, 0))
_full = lambda s: pl.BlockSpec(s, lambda i: (0,) * len(s))

_tc_dense = pl.pallas_call(
    _tc_body,
    grid=(_NBLK,),
    in_specs=[
        _prow_spec,
        _prow_spec,
        _col_spec,
        _col_spec,
        _col_spec,
        _col_spec,
        _full((_D, 2 * _D)),
        _full((1, _D)),
        _full((_D, _D)),
        _full((1, _D)),
        _full((1, 2 * _D)),
        _full((1, 1)),
        _full((1, 1)),
    ],
    out_specs=_col_spec,
    out_shape=jax.ShapeDtypeStruct((_B, 1), jnp.float32),
)


def kernel(d0, d1, d2, d3, d4, user_id, item_id, user_bias, item_bias,
           global_bias, gmf_user_emb, gmf_item_emb, mlp_user_emb, mlp_item_emb,
           W1, b1, W2, b2, Wf, bf):
    gut, mut = gmf_user_emb.T, mlp_user_emb.T
    git, mit = gmf_item_emb.T, mlp_item_emb.T
    p_u = _pack(gut, mut, gut, mut)
    p_i = _pack(git, mit, git, mit)
    o_u, o_i, ub, ib = _make_sc_gather()(user_id, item_id, p_u, p_i,
                                         user_bias, item_bias)
    out = _tc_dense(
        o_u, o_i, user_id.reshape(_B, 1), item_id.reshape(_B, 1),
        ub.reshape(_B, 1), ib.reshape(_B, 1),
        W1, b1.reshape(1, _D), W2, b2.reshape(1, _D), Wf,
        global_bias.reshape(1, 1), bf.reshape(1, 1))
    return out[:, 0]


# speedup vs baseline: 1.6463x; 1.0228x over previous
import functools

import jax
import jax.numpy as jnp
from jax import lax
from jax.experimental import pallas as pl
from jax.experimental.pallas import tpu as pltpu
from jax.experimental.pallas import tpu_sc as plsc

_NC, _NS = 2, 16
_NW = _NC * _NS
_B = 16384
_D = 32
_U = 100001
_HALF = 50176
_PBLK = 512
_NPB = _HALF // _PBLK
_BROWS = 784
_BPW = _B // _NW
_CHUNK = 128
_NCH = _BPW // _CHUNK


def _pack_body(gl, ml, gh, mh, out):
    eye = jnp.eye(_D, dtype=jnp.float32)
    t = lambda x: lax.dot_general(x[...], eye, (((0,), (0,)), ((), ())),
                                  preferred_element_type=jnp.float32)
    out[...] = jnp.concatenate([t(gl), t(ml), t(gh), t(mh)], axis=1)


_pack_lo = pl.BlockSpec((_D, _PBLK), lambda i: (0, i))
_pack_hi = pl.BlockSpec((_D, _PBLK), lambda i: (0, i + _NPB))

_pack = pl.pallas_call(
    _pack_body,
    grid=(_NPB,),
    in_specs=[_pack_lo, _pack_lo, _pack_hi, _pack_hi],
    out_specs=pl.BlockSpec((_PBLK, 128), lambda i: (i, 0)),
    out_shape=jax.ShapeDtypeStruct((_HALF, 128), jnp.float32),
)


def _sc_body(uid, iid, p_u, p_i, b_u, b_i,
             o_u, o_i, o_bs,
             idx, idx2, buf, bias_v, bias_u, sem, sem2):
    cid = lax.axis_index("c")
    sid = lax.axis_index("s")
    base = (cid * _NS + sid) * _BPW

    for ids_hbm, p, b, o_rows, bdst in ((uid, p_u, b_u, o_u, bias_u),
                                        (iid, p_i, b_i, o_i, bias_v)):
        for c in range(_NCH):
            pltpu.sync_copy(ids_hbm.at[pl.ds(base + c * _CHUNK, _CHUNK)],
                            idx.at[c])
        for c in range(_NCH):
            for k in range(_CHUNK // 16):
                sl = pl.ds(16 * k, 16)
                v = idx[c, sl]
                idx2[c, sl] = jnp.where(v < _HALF, v, v - _HALF)
        gs = [pltpu.async_copy(p.at[idx2.at[c]], buf.at[c], sem)
              for c in range(_NCH)]
        bs = [pltpu.async_copy(b.at[idx.at[c]], bdst.at[c], sem)
              for c in range(_NCH)]
        ws = []
        for c in range(_NCH):
            gs[c].wait()
            ws.append(pltpu.async_copy(
                buf.at[c], o_rows.at[pl.ds(base + c * _CHUNK, _CHUNK)], sem2))
        for b_h in bs:
            b_h.wait()
        for w in ws:
            w.wait()
    for c in range(_NCH):
        for k in range(_CHUNK // 16):
            sl = pl.ds(16 * k, 16)
            bias_v[c, sl] = bias_v[c, sl] + bias_u[c, sl]
    for c in range(_NCH):
        pltpu.sync_copy(bias_v.at[c],
                        o_bs.at[pl.ds(base + c * _CHUNK, _CHUNK)])


@functools.cache
def _make_sc_gather():
    return pl.kernel(
        _sc_body,
        out_type=[
            jax.ShapeDtypeStruct((_B, 128), jnp.float32),
            jax.ShapeDtypeStruct((_B, 128), jnp.float32),
            jax.ShapeDtypeStruct((_B,), jnp.float32),
        ],
        mesh=plsc.VectorSubcoreMesh(
            core_axis_name="c", subcore_axis_name="s",
            num_cores=_NC, num_subcores=_NS),
        scratch_types=[
            pltpu.VMEM((_NCH, _CHUNK), jnp.int32),
            pltpu.VMEM((_NCH, _CHUNK), jnp.int32),
            pltpu.VMEM((_NCH, _CHUNK, 128), jnp.float32),
            pltpu.VMEM((_NCH, _CHUNK), jnp.float32),
            pltpu.VMEM((_NCH, _CHUNK), jnp.float32),
            pltpu.SemaphoreType.DMA,
            pltpu.SemaphoreType.DMA,
        ],
        compiler_params=pltpu.CompilerParams(use_tc_tiling_on_sc=True,
                                             needs_layout_passes=False),
    )


_BLK = 2048
_NBLK = _B // _BLK


def _mm(a, b):
    return lax.dot_general(a, b, (((1,), (1,)), ((), ())),
                           preferred_element_type=jnp.float32)


def _tc_body(pu, pi, uidr, iidr, bs, w1, b1, w2, b2, wf, gb, bfs, out):
    pur = pu[...]
    pir = pi[...]
    su = uidr[...] < _HALF
    si = iidr[...] < _HALF
    gu = jnp.where(su, pur[:, 0:_D], pur[:, 64:64 + _D])
    mu = jnp.where(su, pur[:, _D:2 * _D], pur[:, 64 + _D:128])
    gi = jnp.where(si, pir[:, 0:_D], pir[:, 64:64 + _D])
    mi = jnp.where(si, pir[:, _D:2 * _D], pir[:, 64 + _D:128])
    w1v = w1[...]
    wfv = wf[...]
    h = _mm(mu, w1v[:, :_D]) + _mm(mi, w1v[:, _D:]) + b1[...]
    h = jnp.maximum(h, 0.0)
    h = _mm(h, w2[...]) + b2[...]
    r = jnp.sum(gu * gi * wfv[:, :_D], axis=1, keepdims=True)
    r = r + jnp.sum(h * wfv[:, _D:], axis=1, keepdims=True)
    out[...] = bs[...] + r + (gb[0, 0] + bfs[0, 0])


_prow_spec = pl.BlockSpec((_BLK, 128), lambda i: (i, 0))
_col_spec = pl.BlockSpec((_BLK, 1), lambda i: (i, 0))
_full = lambda s: pl.BlockSpec(s, lambda i: (0,) * len(s))

_tc_dense = pl.pallas_call(
    _tc_body,
    grid=(_NBLK,),
    in_specs=[
        _prow_spec,
        _prow_spec,
        _col_spec,
        _col_spec,
        _col_spec,
        _full((_D, 2 * _D)),
        _full((1, _D)),
        _full((_D, _D)),
        _full((1, _D)),
        _full((1, 2 * _D)),
        _full((1, 1)),
        _full((1, 1)),
    ],
    out_specs=_col_spec,
    out_shape=jax.ShapeDtypeStruct((_B, 1), jnp.float32),
)


def kernel(d0, d1, d2, d3, d4, user_id, item_id, user_bias, item_bias,
           global_bias, gmf_user_emb, gmf_item_emb, mlp_user_emb, mlp_item_emb,
           W1, b1, W2, b2, Wf, bf):
    gut, mut = gmf_user_emb.T, mlp_user_emb.T
    git, mit = gmf_item_emb.T, mlp_item_emb.T
    p_u = _pack(gut, mut, gut, mut)
    p_i = _pack(git, mit, git, mit)
    o_u, o_i, bsum = _make_sc_gather()(user_id, item_id, p_u, p_i,
                                       user_bias, item_bias)
    out = _tc_dense(
        o_u, o_i, user_id.reshape(_B, 1), item_id.reshape(_B, 1),
        bsum.reshape(_B, 1),
        W1, b1.reshape(1, _D), W2, b2.reshape(1, _D), Wf,
        global_bias.reshape(1, 1), bf.reshape(1, 1))
    return out[:, 0]


# speedup vs baseline: 2.0546x; 1.2480x over previous
import functools

import jax
import jax.numpy as jnp
from jax import lax
from jax.experimental import pallas as pl
from jax.experimental.pallas import tpu as pltpu
from jax.experimental.pallas import tpu_sc as plsc

_NC, _NS = 2, 16
_NW = _NC * _NS
_B = 16384
_D = 32
_U = 100001
_HALF = 50176
_PBLK = 1024
_NPB = _HALF // _PBLK
_BROWS = 784
_BPW = _B // _NW
_CHUNK = 128
_NCH = _BPW // _CHUNK


def _pack_body(gl, ml, gh, mh, out):
    eye = jnp.eye(_D, dtype=jnp.float32)
    t = lambda x: lax.dot_general(x[...], eye, (((0,), (0,)), ((), ())),
                                  preferred_element_type=jnp.float32)
    out[...] = jnp.concatenate([t(gl), t(ml), t(gh), t(mh)], axis=1)


_pack_lo = pl.BlockSpec((_D, _PBLK), lambda i: (0, i))
_pack_hi = pl.BlockSpec((_D, _PBLK), lambda i: (0, i + _NPB))

_pack = pl.pallas_call(
    _pack_body,
    grid=(_NPB,),
    in_specs=[_pack_lo, _pack_lo, _pack_hi, _pack_hi],
    out_specs=pl.BlockSpec((_PBLK, 128), lambda i: (i, 0)),
    out_shape=jax.ShapeDtypeStruct((_HALF, 128), jnp.float32),
)


def _sc_body(uid, iid, p_u, p_i, b_u, b_i,
             o_u, o_i, o_bs,
             idx, idx2, buf, bias_v, bias_u, sem, sem2):
    cid = lax.axis_index("c")
    sid = lax.axis_index("s")
    base = (cid * _NS + sid) * _BPW

    for ids_hbm, p, b, o_rows, bdst in ((uid, p_u, b_u, o_u, bias_u),
                                        (iid, p_i, b_i, o_i, bias_v)):
        for c in range(_NCH):
            pltpu.sync_copy(ids_hbm.at[pl.ds(base + c * _CHUNK, _CHUNK)],
                            idx.at[c])
        for c in range(_NCH):
            for k in range(_CHUNK // 16):
                sl = pl.ds(16 * k, 16)
                v = idx[c, sl]
                idx2[c, sl] = jnp.where(v < _HALF, v, v - _HALF)
        gs = [pltpu.async_copy(p.at[idx2.at[c]], buf.at[c], sem)
              for c in range(_NCH)]
        bs = [pltpu.async_copy(b.at[idx.at[c]], bdst.at[c], sem)
              for c in range(_NCH)]
        ws = []
        for c in range(_NCH):
            gs[c].wait()
            ws.append(pltpu.async_copy(
                buf.at[c], o_rows.at[pl.ds(base + c * _CHUNK, _CHUNK)], sem2))
        for b_h in bs:
            b_h.wait()
        for w in ws:
            w.wait()
    for c in range(_NCH):
        for k in range(_CHUNK // 16):
            sl = pl.ds(16 * k, 16)
            bias_v[c, sl] = bias_v[c, sl] + bias_u[c, sl]
    for c in range(_NCH):
        pltpu.sync_copy(bias_v.at[c],
                        o_bs.at[pl.ds(base + c * _CHUNK, _CHUNK)])


@functools.cache
def _make_sc_gather():
    return pl.kernel(
        _sc_body,
        out_type=[
            jax.ShapeDtypeStruct((_B, 128), jnp.float32),
            jax.ShapeDtypeStruct((_B, 128), jnp.float32),
            jax.ShapeDtypeStruct((_B,), jnp.float32),
        ],
        mesh=plsc.VectorSubcoreMesh(
            core_axis_name="c", subcore_axis_name="s",
            num_cores=_NC, num_subcores=_NS),
        scratch_types=[
            pltpu.VMEM((_NCH, _CHUNK), jnp.int32),
            pltpu.VMEM((_NCH, _CHUNK), jnp.int32),
            pltpu.VMEM((_NCH, _CHUNK, 128), jnp.float32),
            pltpu.VMEM((_NCH, _CHUNK), jnp.float32),
            pltpu.VMEM((_NCH, _CHUNK), jnp.float32),
            pltpu.SemaphoreType.DMA,
            pltpu.SemaphoreType.DMA,
        ],
        compiler_params=pltpu.CompilerParams(use_tc_tiling_on_sc=True,
                                             needs_layout_passes=False),
    )


_BLK = 2048
_NBLK = _B // _BLK


def _mm(a, b):
    return lax.dot_general(a, b, (((1,), (1,)), ((), ())),
                           preferred_element_type=jnp.float32)


def _tc_body(pu, pi, uidr, iidr, bs, w1, b1, w2, b2, wf, gb, bfs, out):
    pur = pu[...]
    pir = pi[...]
    su = uidr[...] < _HALF
    si = iidr[...] < _HALF
    gu = jnp.where(su, pur[:, 0:_D], pur[:, 64:64 + _D])
    mu = jnp.where(su, pur[:, _D:2 * _D], pur[:, 64 + _D:128])
    gi = jnp.where(si, pir[:, 0:_D], pir[:, 64:64 + _D])
    mi = jnp.where(si, pir[:, _D:2 * _D], pir[:, 64 + _D:128])
    w1v = w1[...]
    wfv = wf[...]
    h = _mm(mu, w1v[:, :_D]) + _mm(mi, w1v[:, _D:]) + b1[...]
    h = jnp.maximum(h, 0.0)
    h = _mm(h, w2[...]) + b2[...]
    r = jnp.sum(gu * gi * wfv[:, :_D], axis=1, keepdims=True)
    r = r + jnp.sum(h * wfv[:, _D:], axis=1, keepdims=True)
    out[...] = bs[...] + r + (gb[0, 0] + bfs[0, 0])


_prow_spec = pl.BlockSpec((_BLK, 128), lambda i: (i, 0))
_col_spec = pl.BlockSpec((_BLK, 1), lambda i: (i, 0))
_full = lambda s: pl.BlockSpec(s, lambda i: (0,) * len(s))

_tc_dense = pl.pallas_call(
    _tc_body,
    grid=(_NBLK,),
    in_specs=[
        _prow_spec,
        _prow_spec,
        _col_spec,
        _col_spec,
        _col_spec,
        _full((_D, 2 * _D)),
        _full((1, _D)),
        _full((_D, _D)),
        _full((1, _D)),
        _full((1, 2 * _D)),
        _full((1, 1)),
        _full((1, 1)),
    ],
    out_specs=_col_spec,
    out_shape=jax.ShapeDtypeStruct((_B, 1), jnp.float32),
)


def kernel(d0, d1, d2, d3, d4, user_id, item_id, user_bias, item_bias,
           global_bias, gmf_user_emb, gmf_item_emb, mlp_user_emb, mlp_item_emb,
           W1, b1, W2, b2, Wf, bf):
    gut, mut = gmf_user_emb.T, mlp_user_emb.T
    git, mit = gmf_item_emb.T, mlp_item_emb.T
    p_u = _pack(gut, mut, gut, mut)
    p_i = _pack(git, mit, git, mit)
    o_u, o_i, bsum = _make_sc_gather()(user_id, item_id, p_u, p_i,
                                       user_bias, item_bias)
    out = _tc_dense(
        o_u, o_i, user_id.reshape(_B, 1), item_id.reshape(_B, 1),
        bsum.reshape(_B, 1),
        W1, b1.reshape(1, _D), W2, b2.reshape(1, _D), Wf,
        global_bias.reshape(1, 1), bf.reshape(1, 1))
    return out[:, 0]
